# baseline (speedup 1.0000x reference)
import functools
import math

import jax
import jax.numpy as jnp
import numpy as np
from jax import lax
from jax.experimental import pallas as pl
from jax.experimental.pallas import tpu as pltpu
from jax.experimental.pallas import tpu_sc as plsc

_P = 4
_L = 16
_HALVES = 2


def _make_pe(s, e):
    ch = np.arange(e, dtype=np.float64)[:, None]
    pos = np.arange(s, dtype=np.float64)[None, :]
    angle = pos * np.power(10000.0, -2.0 * ch / float(e))
    pe = np.where((np.arange(e) % 2 == 0)[:, None], np.sin(angle), np.cos(angle))
    pe = np.broadcast_to(pe.astype(np.float32)[:, :, None], (e, s, _L))
    return jnp.asarray(pe)


def _make_sc_kernel(nw, nc, b, s, e, v, ch0):
    mesh = plsc.VectorSubcoreMesh(core_axis_name="c", subcore_axis_name="s")
    n_chunks = s // _P

    @functools.partial(
        pl.kernel,
        mesh=mesh,
        compiler_params=pltpu.CompilerParams(
            use_tc_tiling_on_sc=False, needs_layout_passes=False),
        out_type=(),
        scratch_types=[
            pltpu.VMEM((v,), jnp.float32),
            pltpu.VMEM((s, _L), jnp.float32),
            pltpu.VMEM((_P, b), jnp.int32),
            pltpu.VMEM((_P, b), jnp.int32),
            pltpu.VMEM((_P, b // 128, 128), jnp.float32),
            pltpu.VMEM((_P, b // 128, 128), jnp.float32),
            pltpu.SemaphoreType.DMA,
            pltpu.SemaphoreType.DMA,
            pltpu.SemaphoreType.DMA,
            pltpu.SemaphoreType.DMA,
        ],
    )
    def sc_kernel(tbl_hbm, idx_hbm, pe_hbm, out_hbm, col_v, pe_v,
                  idx0, idx1, out0, out1, isem0, isem1, wsem0, wsem1):
        idx_v = (idx0, idx1)
        out_v = (out0, out1)
        isem = (isem0, isem1)
        wsem = (wsem0, wsem1)

        tid = lax.axis_index("s") * nc + lax.axis_index("c")
        ch = ch0 + tid
        tr = ch // 8
        r = ch % 8

        def idx_start(c, bi):
            pltpu.async_copy(idx_hbm.at[pl.ds(c * _P, _P)], idx_v[bi], isem[bi])

        def idx_wait(c, bi):
            pltpu.make_async_copy(
                idx_hbm.at[pl.ds(c * _P, _P)], idx_v[bi], isem[bi]).wait()

        def write_start(c, bi):
            pltpu.async_copy(
                out_v[bi], out_hbm.at[pl.ds(c * _P, _P), tr, :, r, :], wsem[bi])

        def write_wait(c, bi):
            pltpu.make_async_copy(
                out_v[bi], out_hbm.at[pl.ds(c * _P, _P), tr, :, r, :],
                wsem[bi]).wait()

        def process(c, bi):
            for sp in range(_P):
                pe16 = pe_v[c * _P + sp, :]

                @plsc.parallel_loop(0, b, step=_L, unroll=8)
                def _(i):
                    iv = idx_v[bi][sp, pl.ds(i, _L)]
                    vals = plsc.load_gather(col_v, [iv])
                    out_v[bi][sp, i // 128, pl.ds(i % 128, _L)] = vals + pe16

        pltpu.sync_copy(tbl_hbm.at[tid], col_v)
        pltpu.sync_copy(pe_hbm.at[tid], pe_v)

        idx_start(0, 0)
        idx_start(1, 1)

        for c in (0, 1):
            bi = c
            idx_wait(c, bi)
            process(c, bi)
            idx_start(c + 2, bi)
            write_start(c, bi)

        @pl.loop(2, n_chunks - 2, step=2)
        def _(g):
            for bi in range(2):
                c = g + bi
                idx_wait(c, bi)
                write_wait(c - 2, bi)
                process(c, bi)
                idx_start(c + 2, bi)
                write_start(c, bi)

        for c in (n_chunks - 2, n_chunks - 1):
            bi = c % 2
            idx_wait(c, bi)
            write_wait(c - 2, bi)
            process(c, bi)
            write_start(c, bi)

        write_wait(n_chunks - 2, (n_chunks - 2) % 2)
        write_wait(n_chunks - 1, (n_chunks - 1) % 2)

    return sc_kernel


def kernel(inputs, table):
    b, s = inputs.shape
    v, e = table.shape
    info = plsc.get_sparse_core_info()
    nc, ns = info.num_cores, info.num_subcores
    nw = nc * ns
    ch_per_call = e // _HALVES

    tbl_t = table.T
    idx_t = inputs.astype(jnp.int32).T
    pe = _make_pe(s, e)

    out_ref = jax.new_ref(jnp.zeros((s, e // 8, b // 128, 8, 128), jnp.float32))
    for h in range(_HALVES):
        ch0 = h * ch_per_call
        tbl_h = tbl_t[ch0:ch0 + ch_per_call]
        pe_h = pe[ch0:ch0 + ch_per_call]
        _make_sc_kernel(nw, nc, b, s, e, v, ch0)(tbl_h, idx_t, pe_h, out_ref)
    out2 = out_ref[...]

    out_t = out2.transpose(0, 1, 3, 2, 4).reshape(s, e, b)
    return out_t.transpose(2, 0, 1)


# speedup vs baseline: 1.0728x; 1.0728x over previous
import functools
import math

import jax
import jax.numpy as jnp
import numpy as np
from jax import lax
from jax.experimental import pallas as pl
from jax.experimental.pallas import tpu as pltpu
from jax.experimental.pallas import tpu_sc as plsc

_P = 4
_L = 16
_HALVES = 2


def _make_pe(s, e):
    ch = np.arange(e, dtype=np.float64)[:, None]
    pos = np.arange(s, dtype=np.float64)[None, :]
    angle = pos * np.power(10000.0, -2.0 * ch / float(e))
    pe = np.where((np.arange(e) % 2 == 0)[:, None], np.sin(angle), np.cos(angle))
    pe = np.broadcast_to(pe.astype(np.float32)[:, :, None], (e, s, _L))
    return jnp.asarray(pe)


def _make_sc_kernel(nw, nc, b, s, e, v, ch0, out_type):
    mesh = plsc.VectorSubcoreMesh(core_axis_name="c", subcore_axis_name="s")
    n_chunks = s // _P

    @functools.partial(
        pl.kernel,
        mesh=mesh,
        compiler_params=pltpu.CompilerParams(
            use_tc_tiling_on_sc=False, needs_layout_passes=False),
        out_type=out_type,
        scratch_types=[
            pltpu.VMEM((v,), jnp.float32),
            pltpu.VMEM((s, _L), jnp.float32),
            pltpu.VMEM((_P, b), jnp.int32),
            pltpu.VMEM((_P, b), jnp.int32),
            pltpu.VMEM((_P, b // 128, 128), jnp.float32),
            pltpu.VMEM((_P, b // 128, 128), jnp.float32),
            pltpu.SemaphoreType.DMA,
            pltpu.SemaphoreType.DMA,
            pltpu.SemaphoreType.DMA,
            pltpu.SemaphoreType.DMA,
        ],
    )
    def sc_kernel(tbl_hbm, idx_hbm, pe_hbm, out_hbm, col_v, pe_v,
                  idx0, idx1, out0, out1, isem0, isem1, wsem0, wsem1):
        idx_v = (idx0, idx1)
        out_v = (out0, out1)
        isem = (isem0, isem1)
        wsem = (wsem0, wsem1)

        tid = lax.axis_index("s") * nc + lax.axis_index("c")
        ch = ch0 + tid
        tr = ch // 8
        r = ch % 8

        def idx_start(c, bi):
            pltpu.async_copy(idx_hbm.at[pl.ds(c * _P, _P)], idx_v[bi], isem[bi])

        def idx_wait(c, bi):
            pltpu.make_async_copy(
                idx_hbm.at[pl.ds(c * _P, _P)], idx_v[bi], isem[bi]).wait()

        def write_start(c, bi):
            pltpu.async_copy(
                out_v[bi], out_hbm.at[pl.ds(c * _P, _P), tr, :, r, :], wsem[bi])

        def write_wait(c, bi):
            pltpu.make_async_copy(
                out_v[bi], out_hbm.at[pl.ds(c * _P, _P), tr, :, r, :],
                wsem[bi]).wait()

        def process(c, bi):
            for sp in range(_P):
                pe16 = pe_v[c * _P + sp, :]

                @plsc.parallel_loop(0, b, step=_L, unroll=8)
                def _(i):
                    iv = idx_v[bi][sp, pl.ds(i, _L)]
                    vals = plsc.load_gather(col_v, [iv])
                    out_v[bi][sp, i // 128, pl.ds(i % 128, _L)] = vals + pe16

        pltpu.sync_copy(tbl_hbm.at[tid], col_v)
        pltpu.sync_copy(pe_hbm.at[tid], pe_v)

        idx_start(0, 0)
        idx_start(1, 1)

        for c in (0, 1):
            bi = c
            idx_wait(c, bi)
            process(c, bi)
            idx_start(c + 2, bi)
            write_start(c, bi)

        @pl.loop(2, n_chunks - 2, step=2)
        def _(g):
            for bi in range(2):
                c = g + bi
                idx_wait(c, bi)
                write_wait(c - 2, bi)
                process(c, bi)
                idx_start(c + 2, bi)
                write_start(c, bi)

        for c in (n_chunks - 2, n_chunks - 1):
            bi = c % 2
            idx_wait(c, bi)
            write_wait(c - 2, bi)
            process(c, bi)
            write_start(c, bi)

        write_wait(n_chunks - 2, (n_chunks - 2) % 2)
        write_wait(n_chunks - 1, (n_chunks - 1) % 2)

    return sc_kernel


def kernel(inputs, table):
    b, s = inputs.shape
    v, e = table.shape
    info = plsc.get_sparse_core_info()
    nc, ns = info.num_cores, info.num_subcores
    nw = nc * ns
    ch_per_call = e // _HALVES

    tbl_t = table.T
    idx_t = inputs.astype(jnp.int32).T
    pe = _make_pe(s, e)

    out_shape = jax.ShapeDtypeStruct((s, e // 8, b // 128, 8, 128), jnp.float32)
    out0 = _make_sc_kernel(nw, nc, b, s, e, v, 0, out_shape)(
        tbl_t[:ch_per_call], idx_t, pe[:ch_per_call])
    out_ref = jax.new_ref(out0)
    _make_sc_kernel(nw, nc, b, s, e, v, ch_per_call, ())(
        tbl_t[ch_per_call:], idx_t, pe[ch_per_call:], out_ref)
    out2 = out_ref[...]

    out_t = out2.transpose(0, 1, 3, 2, 4).reshape(s, e, b)
    return out_t.transpose(2, 0, 1)


# speedup vs baseline: 1.1459x; 1.0681x over previous
import functools
import math

import jax
import jax.numpy as jnp
import numpy as np
from jax import lax
from jax.experimental import pallas as pl
from jax.experimental.pallas import tpu as pltpu
from jax.experimental.pallas import tpu_sc as plsc

_P = 4
_L = 16


def _make_pe(s, e):
    ch = np.arange(e, dtype=np.float64)[:, None]
    pos = np.arange(s, dtype=np.float64)[None, :]
    angle = pos * np.power(10000.0, -2.0 * ch / float(e))
    pe = np.where((np.arange(e) % 2 == 0)[:, None], np.sin(angle), np.cos(angle))
    pe = np.broadcast_to(pe.astype(np.float32)[:, :, None], (e, s, _L))
    return jnp.asarray(pe)


def _make_sc_kernel(nw, nc, b, s, e, v):
    mesh = plsc.VectorSubcoreMesh(core_axis_name="c", subcore_axis_name="s")
    n_chunks = s // _P
    ch_per_tile = e // nw

    @functools.partial(
        pl.kernel,
        mesh=mesh,
        compiler_params=pltpu.CompilerParams(
            use_tc_tiling_on_sc=False, needs_layout_passes=False),
        out_type=jax.ShapeDtypeStruct((s, e // 8, b // 128, 8, 128), jnp.float32),
        scratch_types=[
            pltpu.VMEM((v,), jnp.float32),
            pltpu.VMEM((s, _L), jnp.float32),
            pltpu.VMEM((_P, b), jnp.int32),
            pltpu.VMEM((_P, b), jnp.int32),
            pltpu.VMEM((_P, b // 128, 128), jnp.float32),
            pltpu.VMEM((_P, b // 128, 128), jnp.float32),
            pltpu.SemaphoreType.DMA,
            pltpu.SemaphoreType.DMA,
            pltpu.SemaphoreType.DMA,
            pltpu.SemaphoreType.DMA,
        ],
    )
    def sc_kernel(tbl_hbm, idx_hbm, pe_hbm, out_hbm, col_v, pe_v,
                  idx0, idx1, out0, out1, isem0, isem1, wsem0, wsem1):
        idx_v = (idx0, idx1)
        out_v = (out0, out1)
        isem = (isem0, isem1)
        wsem = (wsem0, wsem1)

        tid = lax.axis_index("s") * nc + lax.axis_index("c")

        def idx_start(c, bi):
            pltpu.async_copy(idx_hbm.at[pl.ds(c * _P, _P)], idx_v[bi], isem[bi])

        def idx_wait(c, bi):
            pltpu.make_async_copy(
                idx_hbm.at[pl.ds(c * _P, _P)], idx_v[bi], isem[bi]).wait()

        def write_start(c, bi, tr, r):
            pltpu.async_copy(
                out_v[bi], out_hbm.at[pl.ds(c * _P, _P), tr, :, r, :], wsem[bi])

        def write_wait(c, bi, tr, r):
            pltpu.make_async_copy(
                out_v[bi], out_hbm.at[pl.ds(c * _P, _P), tr, :, r, :],
                wsem[bi]).wait()

        def process(c, bi):
            for sp in range(_P):
                pe16 = pe_v[c * _P + sp, :]

                @plsc.parallel_loop(0, b, step=_L, unroll=16)
                def _(i):
                    iv = idx_v[bi][sp, pl.ds(i, _L)]
                    vals = plsc.load_gather(col_v, [iv])
                    out_v[bi][sp, i // 128, pl.ds(i % 128, _L)] = vals + pe16

        for cpass in range(ch_per_tile):
            ch = tid * ch_per_tile + cpass
            tr = ch // 8
            r = ch % 8
            pltpu.sync_copy(tbl_hbm.at[ch], col_v)
            pltpu.sync_copy(pe_hbm.at[ch], pe_v)

            idx_start(0, 0)
            idx_start(1, 1)

            for c in (0, 1):
                bi = c
                idx_wait(c, bi)
                process(c, bi)
                idx_start(c + 2, bi)
                write_start(c, bi, tr, r)

            @pl.loop(2, n_chunks - 2, step=2)
            def _(g):
                for bi in range(2):
                    c = g + bi
                    idx_wait(c, bi)
                    write_wait(c - 2, bi, tr, r)
                    process(c, bi)
                    idx_start(c + 2, bi)
                    write_start(c, bi, tr, r)

            for c in (n_chunks - 2, n_chunks - 1):
                bi = c % 2
                idx_wait(c, bi)
                write_wait(c - 2, bi, tr, r)
                process(c, bi)
                write_start(c, bi, tr, r)

            write_wait(n_chunks - 2, (n_chunks - 2) % 2, tr, r)
            write_wait(n_chunks - 1, (n_chunks - 1) % 2, tr, r)

    return sc_kernel


def kernel(inputs, table):
    b, s = inputs.shape
    v, e = table.shape
    info = plsc.get_sparse_core_info()
    nc, ns = info.num_cores, info.num_subcores
    nw = nc * ns

    tbl_t = table.T
    idx_t = inputs.astype(jnp.int32).T
    pe = _make_pe(s, e)
    out2 = _make_sc_kernel(nw, nc, b, s, e, v)(tbl_t, idx_t, pe)
    out_t = out2.transpose(0, 1, 3, 2, 4).reshape(s, e, b)
    return out_t.transpose(2, 0, 1)


# speedup vs baseline: 1.1874x; 1.0362x over previous
import functools
import math

import jax
import jax.numpy as jnp
import numpy as np
from jax import lax
from jax.experimental import pallas as pl
from jax.experimental.pallas import tpu as pltpu
from jax.experimental.pallas import tpu_sc as plsc

_P = 5
_L = 16


def _make_pe(s, e):
    ch = np.arange(e, dtype=np.float64)[:, None]
    pos = np.arange(s, dtype=np.float64)[None, :]
    angle = pos * np.power(10000.0, -2.0 * ch / float(e))
    pe = np.where((np.arange(e) % 2 == 0)[:, None], np.sin(angle), np.cos(angle))
    pe = np.broadcast_to(pe.astype(np.float32)[:, :, None], (e, s, _L))
    return jnp.asarray(pe)


def _make_sc_kernel(nw, nc, b, s, e, v):
    mesh = plsc.VectorSubcoreMesh(core_axis_name="c", subcore_axis_name="s")
    n_chunks = s // _P
    ch_per_tile = e // nw

    @functools.partial(
        pl.kernel,
        mesh=mesh,
        compiler_params=pltpu.CompilerParams(
            use_tc_tiling_on_sc=False, needs_layout_passes=False),
        out_type=jax.ShapeDtypeStruct((s, e // 8, b // 128, 8, 128), jnp.float32),
        scratch_types=[
            pltpu.VMEM((v,), jnp.float32),
            pltpu.VMEM((s, _L), jnp.float32),
            pltpu.VMEM((_P, b), jnp.int32),
            pltpu.VMEM((_P, b), jnp.int32),
            pltpu.VMEM((_P, b // 128, 128), jnp.float32),
            pltpu.VMEM((_P, b // 128, 128), jnp.float32),
            pltpu.SemaphoreType.DMA,
            pltpu.SemaphoreType.DMA,
            pltpu.SemaphoreType.DMA,
            pltpu.SemaphoreType.DMA,
        ],
    )
    def sc_kernel(tbl_hbm, idx_hbm, pe_hbm, out_hbm, col_v, pe_v,
                  idx0, idx1, out0, out1, isem0, isem1, wsem0, wsem1):
        idx_v = (idx0, idx1)
        out_v = (out0, out1)
        isem = (isem0, isem1)
        wsem = (wsem0, wsem1)

        tid = lax.axis_index("s") * nc + lax.axis_index("c")

        def idx_start(c, bi):
            pltpu.async_copy(idx_hbm.at[pl.ds(c * _P, _P)], idx_v[bi], isem[bi])

        def idx_wait(c, bi):
            pltpu.make_async_copy(
                idx_hbm.at[pl.ds(c * _P, _P)], idx_v[bi], isem[bi]).wait()

        def write_start(c, bi, tr, r):
            pltpu.async_copy(
                out_v[bi], out_hbm.at[pl.ds(c * _P, _P), tr, :, r, :], wsem[bi])

        def write_wait(c, bi, tr, r):
            pltpu.make_async_copy(
                out_v[bi], out_hbm.at[pl.ds(c * _P, _P), tr, :, r, :],
                wsem[bi]).wait()

        def process(c, bi):
            for sp in range(_P):
                pe16 = pe_v[c * _P + sp, :]

                @plsc.parallel_loop(0, b, step=_L, unroll=8)
                def _(i):
                    iv = idx_v[bi][sp, pl.ds(i, _L)]
                    vals = plsc.load_gather(col_v, [iv])
                    out_v[bi][sp, i // 128, pl.ds(i % 128, _L)] = vals + pe16

        for cpass in range(ch_per_tile):
            ch = tid * ch_per_tile + cpass
            tr = ch // 8
            r = ch % 8
            pltpu.sync_copy(tbl_hbm.at[ch], col_v)
            pltpu.sync_copy(pe_hbm.at[ch], pe_v)

            idx_start(0, 0)
            idx_start(1, 1)

            for c in (0, 1):
                bi = c
                idx_wait(c, bi)
                process(c, bi)
                idx_start(c + 2, bi)
                write_start(c, bi, tr, r)

            @pl.loop(2, n_chunks - 2, step=2)
            def _(g):
                for bi in range(2):
                    c = g + bi
                    idx_wait(c, bi)
                    write_wait(c - 2, bi, tr, r)
                    process(c, bi)
                    idx_start(c + 2, bi)
                    write_start(c, bi, tr, r)

            for c in (n_chunks - 2, n_chunks - 1):
                bi = c % 2
                idx_wait(c, bi)
                write_wait(c - 2, bi, tr, r)
                process(c, bi)
                write_start(c, bi, tr, r)

            write_wait(n_chunks - 2, (n_chunks - 2) % 2, tr, r)
            write_wait(n_chunks - 1, (n_chunks - 1) % 2, tr, r)

    return sc_kernel


def kernel(inputs, table):
    b, s = inputs.shape
    v, e = table.shape
    info = plsc.get_sparse_core_info()
    nc, ns = info.num_cores, info.num_subcores
    nw = nc * ns

    tbl_t = table.T
    idx_t = inputs.astype(jnp.int32).T
    pe = _make_pe(s, e)
    out2 = _make_sc_kernel(nw, nc, b, s, e, v)(tbl_t, idx_t, pe)
    out_t = out2.transpose(0, 1, 3, 2, 4).reshape(s, e, b)
    return out_t.transpose(2, 0, 1)
